# baseline (speedup 1.0000x reference)
import functools

import jax
import jax.numpy as jnp
from jax import lax
from jax.experimental import pallas as pl
from jax.experimental.pallas import tpu as pltpu
from jax.experimental.pallas import tpu_sc as plsc

N_VOCAB = 8
DIM = 10
LANES = 16
NUM_WORKERS = 32
ROWS_PER_CHUNK = 40


def _sc_lookup_kernel(n_rows, n_cols):
    cols_per_w = n_cols // NUM_WORKERS
    n_chunks = n_rows // ROWS_PER_CHUNK
    mesh = plsc.VectorSubcoreMesh(core_axis_name="c", subcore_axis_name="s")

    @functools.partial(
        pl.kernel,
        out_type=jax.ShapeDtypeStruct((n_rows * n_cols,), jnp.float32),
        mesh=mesh,
        scratch_types=[
            pltpu.VMEM((DIM, LANES), jnp.float32),
            pltpu.VMEM((DIM, LANES), jnp.float32),
            pltpu.VMEM((LANES,), jnp.float32),
            pltpu.VMEM((2, ROWS_PER_CHUNK, cols_per_w), jnp.int32),
            pltpu.VMEM((2, ROWS_PER_CHUNK, cols_per_w), jnp.float32),
            pltpu.SemaphoreType.DMA,
            pltpu.SemaphoreType.DMA,
            pltpu.SemaphoreType.DMA,
            pltpu.SemaphoreType.DMA,
        ],
    )
    def body(idx_hbm, tab_hbm, w_hbm, b_hbm, out_hbm,
             tab_v, w_v, b_v, idx_v, out_v,
             sem_in0, sem_in1, sem_out0, sem_out1):
        sem_in = (sem_in0, sem_in1)
        sem_out = (sem_out0, sem_out1)

        wid = lax.axis_index("c") * (NUM_WORKERS // 2) + lax.axis_index("s")
        col0 = wid * cols_per_w

        def start_in(c):
            b = c & 1
            return pltpu.async_copy(
                idx_hbm.at[pl.ds(c * ROWS_PER_CHUNK, ROWS_PER_CHUNK),
                           pl.ds(col0, cols_per_w)],
                idx_v.at[b], sem_in[b])

        in_handles = [None, None]
        in_handles[0] = start_in(0)

        pltpu.sync_copy(tab_hbm, tab_v)
        pltpu.sync_copy(w_hbm, w_v)
        pltpu.sync_copy(b_hbm, b_v)

        acc = b_v[...]
        for d in range(DIM):
            acc = acc + tab_v[d] * w_v[d]
        lut = 1.0 / (1.0 + jnp.exp(-acc))

        def out_descr(c, rr):
            b = c & 1
            return pltpu.make_async_copy(
                out_v.at[b, rr],
                out_hbm.at[pl.ds(
                    (c * ROWS_PER_CHUNK + rr) * n_cols + col0, cols_per_w)],
                sem_out[b])

        def drain_out(c):
            b = c & 1
            pltpu.make_async_copy(
                idx_hbm.at[pl.ds(0, ROWS_PER_CHUNK), pl.ds(0, cols_per_w)],
                idx_v.at[b], sem_out[b]).wait()

        for c in range(n_chunks):
            b = c & 1
            if c + 1 < n_chunks:
                in_handles[1 - b] = start_in(c + 1)
            in_handles[b].wait()
            if c >= 2:
                drain_out(c - 2)

            @plsc.parallel_loop(0, ROWS_PER_CHUNK, step=1, unroll=4)
            def row_body(r, b=b, c=c):
                @plsc.parallel_loop(0, cols_per_w, step=LANES, unroll=8)
                def vec_body(s, r=r, b=b):
                    iv = idx_v[b, r, pl.ds(s, LANES)]
                    out_v[b, r, pl.ds(s, LANES)] = jnp.take_along_axis(
                        lut, iv, axis=0)

                out_descr(c, r).start()

        for c in range(max(0, n_chunks - 2), n_chunks):
            drain_out(c)

    return body


def kernel(indices, table, W, b):
    B, L = indices.shape

    tab_pad = jnp.pad(table.astype(jnp.float32).T,
                      ((0, 0), (0, LANES - N_VOCAB)))
    w_pad = jnp.broadcast_to(
        W.reshape(DIM, 1).astype(jnp.float32), (DIM, LANES))
    b_pad = jnp.broadcast_to(b.astype(jnp.float32), (LANES,))

    idx_t = indices.astype(jnp.int32).T
    out_flat = _sc_lookup_kernel(L, B)(idx_t, tab_pad, w_pad, b_pad)
    return out_flat.reshape(L, B, 1).transpose((1, 0, 2))


# speedup vs baseline: 1.0140x; 1.0140x over previous
import functools

import jax
import jax.numpy as jnp
from jax import lax
from jax.experimental import pallas as pl
from jax.experimental.pallas import tpu as pltpu
from jax.experimental.pallas import tpu_sc as plsc

N_VOCAB = 8
DIM = 10
LANES = 16
NUM_WORKERS = 32
ROWS_PER_CHUNK = 40


def _sc_lookup_kernel(n_rows, n_cols):
    cols_per_w = n_cols // NUM_WORKERS
    n_chunks = n_rows // ROWS_PER_CHUNK
    mesh = plsc.VectorSubcoreMesh(core_axis_name="c", subcore_axis_name="s")

    @functools.partial(
        pl.kernel,
        out_type=jax.ShapeDtypeStruct((n_rows * n_cols,), jnp.float32),
        mesh=mesh,
        scratch_types=[
            pltpu.VMEM((DIM, LANES), jnp.float32),
            pltpu.VMEM((DIM, LANES), jnp.float32),
            pltpu.VMEM((LANES,), jnp.float32),
            pltpu.VMEM((2, ROWS_PER_CHUNK, cols_per_w), jnp.int32),
            pltpu.VMEM((2, ROWS_PER_CHUNK, cols_per_w), jnp.float32),
            pltpu.SemaphoreType.DMA,
            pltpu.SemaphoreType.DMA,
            pltpu.SemaphoreType.DMA,
            pltpu.SemaphoreType.DMA,
        ],
    )
    def body(idx_hbm, tab_hbm, w_hbm, b_hbm, out_hbm,
             tab_v, w_v, b_v, idx_v, out_v,
             sem_in0, sem_in1, sem_out0, sem_out1):
        sem_in = (sem_in0, sem_in1)
        sem_out = (sem_out0, sem_out1)

        wid = lax.axis_index("c") * (NUM_WORKERS // 2) + lax.axis_index("s")
        col0 = wid * cols_per_w

        def start_in(c):
            b = c & 1
            return pltpu.async_copy(
                idx_hbm.at[pl.ds(c * ROWS_PER_CHUNK, ROWS_PER_CHUNK),
                           pl.ds(col0, cols_per_w)],
                idx_v.at[b], sem_in[b])

        in_handles = [None, None]
        in_handles[0] = start_in(0)

        pltpu.sync_copy(tab_hbm, tab_v)
        pltpu.sync_copy(w_hbm, w_v)
        pltpu.sync_copy(b_hbm, b_v)

        acc = b_v[...]
        for d in range(DIM):
            acc = acc + tab_v[d] * w_v[d]
        lut = 1.0 / (1.0 + jnp.exp(-acc))

        def out_descr(c, rr):
            b = c & 1
            return pltpu.make_async_copy(
                out_v.at[b, rr],
                out_hbm.at[pl.ds(
                    (c * ROWS_PER_CHUNK + rr) * n_cols + col0, cols_per_w)],
                sem_out[b])

        def drain_out(c):
            b = c & 1
            pltpu.make_async_copy(
                idx_hbm.at[pl.ds(0, ROWS_PER_CHUNK), pl.ds(0, cols_per_w)],
                idx_v.at[b], sem_out[b]).wait()

        for c in range(n_chunks):
            b = c & 1
            if c + 1 < n_chunks:
                in_handles[1 - b] = start_in(c + 1)
            in_handles[b].wait()
            if c >= 2:
                drain_out(c - 2)

            @plsc.parallel_loop(0, ROWS_PER_CHUNK, step=1, unroll=2)
            def row_body(r, b=b, c=c):
                @plsc.parallel_loop(0, cols_per_w, step=LANES, unroll=8)
                def vec_body(s, r=r, b=b):
                    iv = idx_v[b, r, pl.ds(s, LANES)]
                    out_v[b, r, pl.ds(s, LANES)] = jnp.take_along_axis(
                        lut, iv, axis=0)

                out_descr(c, r).start()

        for c in range(max(0, n_chunks - 2), n_chunks):
            drain_out(c)

    return body


def kernel(indices, table, W, b):
    B, L = indices.shape

    tab_pad = jnp.pad(table.astype(jnp.float32).T,
                      ((0, 0), (0, LANES - N_VOCAB)))
    w_pad = jnp.broadcast_to(
        W.reshape(DIM, 1).astype(jnp.float32), (DIM, LANES))
    b_pad = jnp.broadcast_to(b.astype(jnp.float32), (LANES,))

    idx_t = indices.astype(jnp.int32).T
    out_flat = _sc_lookup_kernel(L, B)(idx_t, tab_pad, w_pad, b_pad)
    return out_flat.reshape(L, B, 1).transpose((1, 0, 2))
